# Initial kernel scaffold; baseline (speedup 1.0000x reference)
import jax
import jax.numpy as jnp
from jax.experimental import pallas as pl


def kernel(zIG, xt_enc, edge_index, Ws, Wd, F2_W1, F2_W2, F1_W1, F1_W2):
    raise NotImplementedError("write your pallas kernel here")


# speedup vs baseline: 1.4206x; 1.4206x over previous
import jax
import jax.numpy as jnp
from jax import lax
from jax.experimental import pallas as pl

N = 10000
E = 320000
H = 128
Z = 64
NP = 10112
BN = 1264
BE = 4000
PREC = lax.Precision.HIGHEST


def _dot(a, b, ca, cb):
    return lax.dot_general(a, b, (((ca,), (cb,)), ((), ())), precision=PREC)


def _pre_body(z_ref, x_ref, ws_ref, wd_ref, w1_ref, p_ref, as_ref, ad_ref):
    m0 = _dot(ws_ref[0], wd_ref[0], 0, 0)
    m1 = _dot(ws_ref[1], wd_ref[1], 0, 0)
    z = z_ref[...]
    x = x_ref[...]
    w1 = w1_ref[...]
    p_ref[...] = jnp.concatenate([_dot(z, m0, 1, 0), _dot(z, m1, 1, 0)], axis=1)
    as_ref[...] = _dot(x, w1[:, :H], 1, 1)
    ad_ref[...] = _dot(x, w1[:, H:], 1, 1)


def _tc_pre(zp, xp, Ws, Wd, F2_W1):
    f32 = jnp.float32
    return pl.pallas_call(
        _pre_body,
        grid=(NP // BN,),
        in_specs=[
            pl.BlockSpec((BN, Z), lambda i: (i, 0)),
            pl.BlockSpec((BN, H), lambda i: (i, 0)),
            pl.BlockSpec((2, H, Z), lambda i: (0, 0, 0)),
            pl.BlockSpec((2, H, Z), lambda i: (0, 0, 0)),
            pl.BlockSpec((H, 2 * H), lambda i: (0, 0)),
        ],
        out_specs=[
            pl.BlockSpec((BN, H), lambda i: (i, 0)),
            pl.BlockSpec((BN, H), lambda i: (i, 0)),
            pl.BlockSpec((BN, H), lambda i: (i, 0)),
        ],
        out_shape=[
            jax.ShapeDtypeStruct((NP, H), f32),
            jax.ShapeDtypeStruct((NP, H), f32),
            jax.ShapeDtypeStruct((NP, H), f32),
        ],
    )(zp, xp, Ws, Wd, F2_W1)


def _edge_body(p_ref, zd_ref, as_ref, ad_ref, y0_ref, y1_ref, den_ref):
    p = p_ref[...]
    zd = zd_ref[...]
    e0 = jnp.sum(p[:, :Z] * zd, axis=1, keepdims=True)
    e1 = jnp.sum(p[:, Z:] * zd, axis=1, keepdims=True)
    e0 = jnp.where(e0 > 0.0, e0, 0.01 * e0)
    e1 = jnp.where(e1 > 0.0, e1, 0.01 * e1)
    w0 = jnp.exp(e0)
    w1 = jnp.exp(e1)
    m = jnp.maximum(as_ref[...] + ad_ref[...], 0.0)
    y0_ref[...] = w0 * m
    y1_ref[...] = w1 * m
    den_ref[...] = jnp.concatenate(
        [w0, w1, jnp.ones_like(w0), jnp.zeros((w0.shape[0], 5), w0.dtype)],
        axis=1)


def _tc_edge(pg, zdg, asg, adg):
    f32 = jnp.float32
    return pl.pallas_call(
        _edge_body,
        grid=(E // BE,),
        in_specs=[
            pl.BlockSpec((BE, H), lambda i: (i, 0)),
            pl.BlockSpec((BE, Z), lambda i: (i, 0)),
            pl.BlockSpec((BE, H), lambda i: (i, 0)),
            pl.BlockSpec((BE, H), lambda i: (i, 0)),
        ],
        out_specs=[
            pl.BlockSpec((BE, H), lambda i: (i, 0)),
            pl.BlockSpec((BE, H), lambda i: (i, 0)),
            pl.BlockSpec((BE, 8), lambda i: (i, 0)),
        ],
        out_shape=[
            jax.ShapeDtypeStruct((E, H), f32),
            jax.ShapeDtypeStruct((E, H), f32),
            jax.ShapeDtypeStruct((E, 8), f32),
        ],
    )(pg, zdg, asg, adg)


def _post_body(r0_ref, r1_ref, den_ref, w2_ref, f1a_ref, f1b_ref, o_ref):
    r0 = r0_ref[...]
    r1 = r1_ref[...]
    den = den_ref[...]
    dg = jnp.maximum(den[:, 2:3], 1.0)
    sc0 = 1.0 / (jnp.maximum(den[:, 0:1], 1e-16) * dg)
    sc1 = 1.0 / (jnp.maximum(den[:, 1:2], 1e-16) * dg)
    w2 = w2_ref[...]
    s0 = _dot(r0 * sc0, w2, 1, 1)
    s1 = _dot(r1 * sc1, w2, 1, 1)
    h = jnp.concatenate([s0, s1], axis=1)
    t = jnp.maximum(_dot(h, f1a_ref[...], 1, 1), 0.0)
    o_ref[...] = _dot(t, f1b_ref[...], 1, 1)


def _tc_post(r0, r1, den, F2_W2, F1_W1, F1_W2):
    return pl.pallas_call(
        _post_body,
        grid=(NP // BN,),
        in_specs=[
            pl.BlockSpec((BN, H), lambda i: (i, 0)),
            pl.BlockSpec((BN, H), lambda i: (i, 0)),
            pl.BlockSpec((BN, 8), lambda i: (i, 0)),
            pl.BlockSpec((H, H), lambda i: (0, 0)),
            pl.BlockSpec((H, 2 * H), lambda i: (0, 0)),
            pl.BlockSpec((H, H), lambda i: (0, 0)),
        ],
        out_specs=pl.BlockSpec((BN, H), lambda i: (i, 0)),
        out_shape=jax.ShapeDtypeStruct((NP, H), jnp.float32),
    )(r0, r1, den, F2_W2, F1_W1, F1_W2)


def kernel(zIG, xt_enc, edge_index, Ws, Wd, F2_W1, F2_W2, F1_W1, F1_W2):
    src = edge_index[0].astype(jnp.int32)
    dst = edge_index[1].astype(jnp.int32)
    zp = jnp.pad(zIG, ((0, NP - N), (0, 0)))
    xp = jnp.pad(xt_enc, ((0, NP - N), (0, 0)))
    P, AS, AD = _tc_pre(zp, xp, Ws, Wd, F2_W1)
    pg = jnp.take(P, src, axis=0)
    zdg = jnp.take(zIG, dst, axis=0)
    asg = jnp.take(AS, src, axis=0)
    adg = jnp.take(AD, dst, axis=0)
    y0, y1, den = _tc_edge(pg, zdg, asg, adg)
    r0 = jax.ops.segment_sum(y0, dst, num_segments=NP)
    r1 = jax.ops.segment_sum(y1, dst, num_segments=NP)
    dn = jax.ops.segment_sum(den, dst, num_segments=NP)
    out = _tc_post(r0, r1, dn, F2_W2, F1_W1, F1_W2)
    return out[:N]
